# Initial kernel scaffold; baseline (speedup 1.0000x reference)
import jax
import jax.numpy as jnp
from jax.experimental import pallas as pl


def kernel(x, edge_index, We_s, We_v, M0, b0, M1, b1, M2, b2, M3, b3, Wm_s, Wm_v, Wn_s, Wn_v, Wp_s, Wp_v):
    raise NotImplementedError("write your pallas kernel here")


# speedup vs baseline: 7.0333x; 7.0333x over previous
import functools
import math

import jax
import jax.numpy as jnp
from jax.experimental import pallas as pl

_N = 10000
_E = 320000
_DSC = 125
_DH = 64
_NRB = 4
_NTP = 258
_EB = 1000
_OUTW = 144

_SQRT2 = math.sqrt(2.0)
_SQRT3 = math.sqrt(3.0)


def _edge_body(g_ref, pj_ref, m0_ref, b0_ref, m1_ref, b1_ref, m2_ref, b2_ref,
               m3_ref, b3_ref, wvh_ref, out_ref):
    g = g_ref[...]
    pos_i = g[:, 0:3]
    m_s = g[:, 3:128]
    pj = pj_ref[:, 0:3]
    r = pos_i - pj
    d2 = jnp.sum(r * r, axis=1, keepdims=True) + 1e-12
    d = jnp.sqrt(d2)
    y1 = (_SQRT3 / d) * r
    pid = jnp.pi * d
    rad = (_SQRT2 / d) * jnp.concatenate(
        [jnp.sin(pid), jnp.sin(2.0 * pid), jnp.sin(3.0 * pid), jnp.sin(4.0 * pid)],
        axis=1)
    h = jax.nn.gelu(jnp.dot(rad, m0_ref[...], preferred_element_type=jnp.float32) + b0_ref[...])
    h = jax.nn.gelu(jnp.dot(h, m1_ref[...], preferred_element_type=jnp.float32) + b1_ref[...])
    h = jax.nn.gelu(jnp.dot(h, m2_ref[...], preferred_element_type=jnp.float32) + b2_ref[...])
    mix = jnp.dot(h, m3_ref[...], preferred_element_type=jnp.float32) + b3_ref[...]
    es_main = m_s * mix[:, 0:125]
    piy = jnp.sum(pos_i * y1, axis=1, keepdims=True)
    y1y1 = jnp.sum(y1 * y1, axis=1, keepdims=True)
    es126 = piy * mix[:, 252:253]
    es127 = y1y1 * mix[:, 253:254]
    alpha = (jnp.dot(m_s * mix[:, 126:251], wvh_ref[...],
                     preferred_element_type=jnp.float32)
             + mix[:, 251:252] + mix[:, 255:256])
    uv = y1 * alpha + pos_i * mix[:, 254:255]
    out_ref[...] = jnp.concatenate([
        es_main, mix[:, 125:126], es126, es127, uv,
        jnp.ones((_EB, 1), jnp.float32),
        jnp.zeros((_EB, _OUTW - 132), jnp.float32),
    ], axis=1)


@functools.partial(jax.jit, static_argnums=())
def _edge_stage(g, pj, m0, b0, m1, b1, m2, b2, m3, b3, wvh):
    grid = _E // _EB
    full = lambda shape: pl.BlockSpec(shape, lambda i: (0, 0))
    return pl.pallas_call(
        _edge_body,
        grid=(grid,),
        in_specs=[
            pl.BlockSpec((_EB, 128), lambda i: (i, 0)),
            pl.BlockSpec((_EB, 4), lambda i: (i, 0)),
            full((_NRB, _DH)), full((1, _DH)),
            full((_DH, _DH)), full((1, _DH)),
            full((_DH, _DH)), full((1, _DH)),
            full((_DH, _NTP)), full((1, _NTP)),
            full((125, 1)),
        ],
        out_specs=pl.BlockSpec((_EB, _OUTW), lambda i: (i, 0)),
        out_shape=jax.ShapeDtypeStruct((_E, _OUTW), jnp.float32),
    )(g, pj, m0, b0, m1, b1, m2, b2, m3, b3, wvh)


def kernel(x, edge_index, We_s, We_v, M0, b0, M1, b1, M2, b2, M3, b3,
           Wm_s, Wm_v, Wn_s, Wn_v, Wp_s, Wp_v):
    senders = edge_index[0]
    receivers = edge_index[1]
    T = We_s.shape[0]
    for t in range(T):
        pos = x[:, :3]
        sc = x[:, 3:]
        wev = We_v[t][0, 0]
        wv = Wm_v[t][:, 0]
        colscale = jnp.ones((_NTP,), jnp.float32)
        colscale = colscale.at[251].set(wv[125])
        colscale = colscale.at[252].set(wev / _SQRT3)
        colscale = colscale.at[253].set(1.0 / _SQRT3)
        colscale = colscale.at[254].set(wev * wv[126])
        colscale = colscale.at[255].set(wv[127])
        m0 = M0[t] / math.sqrt(float(_NRB))
        m1 = M1[t] / math.sqrt(float(_DH))
        m2 = M2[t] / math.sqrt(float(_DH))
        m3 = (M3[t] / math.sqrt(float(_DH))) * colscale[None, :]
        b3s = b3[t] * colscale
        wvh = wv[:125][:, None]

        sc_t = sc @ (We_s[t] / math.sqrt(float(_DSC)))
        table = jnp.concatenate([pos, sc_t], axis=1)
        g = jnp.take(table, senders, axis=0)
        pj = jnp.take(jnp.pad(pos, ((0, 0), (0, 1))), receivers, axis=0)

        eout = _edge_stage(g, pj, m0, b0[t][None, :], m1, b1[t][None, :],
                           m2, b2[t][None, :], m3, b3s[None, :], wvh)

        acc = jax.ops.segment_sum(eout, receivers, num_segments=_N)
        cnt = jnp.maximum(acc[:, 131:132], 1.0)
        ms = acc[:, :128] / cnt
        mvc = acc[:, 128:131] / cnt
        u_s = (ms @ (Wm_s[t] / math.sqrt(128.0))
               + sc @ (Wn_s[t] / math.sqrt(float(_DSC))))
        u_v = mvc / math.sqrt(128.0) + pos * Wn_v[t][0, 0]
        gate = jax.nn.sigmoid(u_s[:, _DSC:_DSC + 1])
        new_sc = jax.nn.gelu(u_s[:, :_DSC])
        new_v = u_v * gate
        sc2 = new_sc @ (Wp_s[t] / math.sqrt(float(_DSC)))
        v2 = new_v * Wp_v[t][0, 0]
        x = jnp.concatenate([v2, sc2], axis=1)
    return x


# speedup vs baseline: 9.4701x; 1.3465x over previous
import functools
import math

import jax
import jax.numpy as jnp
from jax import lax
from jax.experimental import pallas as pl
from jax.experimental.pallas import tpu as pltpu
from jax.experimental.pallas import tpu_sc as plsc

_N = 10000
_E = 320000
_DSC = 125
_DH = 64
_NRB = 4
_NTP = 258

_NC = 2
_NS = 16
_NW = _NC * _NS
_CH = 128
_NCHUNK = 80
_EPW = _CH * _NCHUNK
_EPAD = _EPW * _NW
_EB = 1024
_OUTW = 128
_NACC = 10112
_RPT = _NACC // _NS

_SQRT2 = math.sqrt(2.0)
_SQRT3 = math.sqrt(3.0)

_MESH = plsc.VectorSubcoreMesh(core_axis_name="c", subcore_axis_name="s")


def _gather_body(table_ref, sidx_ref, ridx_ref, g_ref, pj_ref,
                 table_sh, sidx_v, ridx_v, rows_v, sem_g):
    cid = lax.axis_index("c")
    sid = lax.axis_index("s")
    wid = sid * _NC + cid
    pltpu.sync_copy(table_ref.at[pl.ds(sid * _RPT, _RPT)],
                    table_sh.at[pl.ds(sid * _RPT, _RPT)])
    plsc.subcore_barrier()
    base = wid * _EPW
    pltpu.sync_copy(sidx_ref.at[wid], sidx_v)
    pltpu.sync_copy(ridx_ref.at[wid], ridx_v)

    def body(j, carry):
        off = base + j * _CH
        pltpu.async_copy(table_sh.at[sidx_v.at[j]], rows_v, sem_g).wait()
        pltpu.sync_copy(rows_v, g_ref.at[pl.ds(off, _CH)])
        pltpu.async_copy(table_sh.at[ridx_v.at[j]], rows_v, sem_g).wait()
        pltpu.sync_copy(rows_v, pj_ref.at[pl.ds(off, _CH)])
        return carry

    lax.fori_loop(0, _NCHUNK, body, 0)


@functools.partial(
    pl.kernel,
    mesh=_MESH,
    out_type=(
        jax.ShapeDtypeStruct((_EPAD, 128), jnp.float32),
        jax.ShapeDtypeStruct((_EPAD, 128), jnp.float32),
    ),
    scratch_types=[
        pltpu.VMEM_SHARED((_NACC, 128), jnp.float32),
        pltpu.VMEM((_NCHUNK, _CH), jnp.int32),
        pltpu.VMEM((_NCHUNK, _CH), jnp.int32),
        pltpu.VMEM((_CH, 128), jnp.float32),
        pltpu.SemaphoreType.DMA,
    ],
)
def _sc_gather(table, sidx3, ridx3, g_out, pj_out,
               table_sh, sidx_v, ridx_v, rows_v, sem_g):
    _gather_body(table, sidx3, ridx3, g_out, pj_out,
                 table_sh, sidx_v, ridx_v, rows_v, sem_g)


def _scatter_body(eout_ref, ridx_ref, zeros_ref, out_ref,
                  idx_v, pay_v, acc_sh, sem):
    cid = lax.axis_index("c")
    sid = lax.axis_index("s")
    wid = sid * _NC + cid
    pltpu.sync_copy(zeros_ref.at[pl.ds(sid * _RPT, _RPT)],
                    acc_sh.at[pl.ds(sid * _RPT, _RPT)])
    plsc.subcore_barrier()
    pltpu.sync_copy(ridx_ref.at[wid], idx_v)
    base = wid * _EPW

    def body(j, carry):
        pltpu.sync_copy(eout_ref.at[pl.ds(base + j * _CH, _CH)], pay_v)
        pltpu.sync_copy(pay_v, acc_sh.at[idx_v.at[j]], add=True)
        return carry

    lax.fori_loop(0, _NCHUNK, body, 0)
    plsc.subcore_barrier()
    pltpu.sync_copy(acc_sh.at[pl.ds(sid * _RPT, _RPT)],
                    out_ref.at[cid, pl.ds(sid * _RPT, _RPT)])


@functools.partial(
    pl.kernel,
    mesh=_MESH,
    out_type=jax.ShapeDtypeStruct((_NC, _NACC, _OUTW), jnp.float32),
    scratch_types=[
        pltpu.VMEM((_NCHUNK, _CH), jnp.int32),
        pltpu.VMEM((_CH, _OUTW), jnp.float32),
        pltpu.VMEM_SHARED((_NACC, _OUTW), jnp.float32),
        pltpu.SemaphoreType.DMA,
    ],
)
def _sc_scatter(eout, ridx3, zeros, partial, idx_v, pay_v, acc_sh, sem):
    _scatter_body(eout, ridx3, zeros, partial, idx_v, pay_v, acc_sh, sem)


def _edge_body(g_ref, pj_ref, m0_ref, b0_ref, m1_ref, b1_ref, m2_ref, b2_ref,
               m3_ref, b3_ref, wvh_ref, out_ref, uv_ref):
    g = g_ref[...]
    pos_i = g[:, 0:3]
    m_s = g[:, 3:128]
    pj = pj_ref[:, 0:3]
    r = pos_i - pj
    d2 = jnp.sum(r * r, axis=1, keepdims=True) + 1e-12
    d = jnp.sqrt(d2)
    y1 = (_SQRT3 / d) * r
    pid = jnp.pi * d
    rad = (_SQRT2 / d) * jnp.concatenate(
        [jnp.sin(pid), jnp.sin(2.0 * pid), jnp.sin(3.0 * pid), jnp.sin(4.0 * pid)],
        axis=1)
    h = jax.nn.gelu(jnp.dot(rad, m0_ref[...], preferred_element_type=jnp.float32) + b0_ref[...])
    h = jax.nn.gelu(jnp.dot(h, m1_ref[...], preferred_element_type=jnp.float32) + b1_ref[...])
    h = jax.nn.gelu(jnp.dot(h, m2_ref[...], preferred_element_type=jnp.float32) + b2_ref[...])
    mix = jnp.dot(h, m3_ref[...], preferred_element_type=jnp.float32) + b3_ref[...]
    es_main = m_s * mix[:, 0:125]
    piy = jnp.sum(pos_i * y1, axis=1, keepdims=True)
    y1y1 = jnp.sum(y1 * y1, axis=1, keepdims=True)
    es126 = piy * mix[:, 252:253]
    es127 = y1y1 * mix[:, 253:254]
    alpha = (jnp.dot(m_s * mix[:, 126:251], wvh_ref[...],
                     preferred_element_type=jnp.float32)
             + mix[:, 251:252] + mix[:, 255:256])
    uv = y1 * alpha + pos_i * mix[:, 254:255]
    out_ref[...] = jnp.concatenate(
        [es_main, mix[:, 125:126], es126, es127], axis=1)
    uv_ref[...] = jnp.concatenate(
        [uv, jnp.ones((_EB, 1), jnp.float32),
         jnp.zeros((_EB, 4), jnp.float32)], axis=1)


def _edge_stage(g, pj, m0, b0, m1, b1, m2, b2, m3, b3, wvh):
    grid = _EPAD // _EB
    full = lambda shape: pl.BlockSpec(shape, lambda i: (0, 0))
    return pl.pallas_call(
        _edge_body,
        grid=(grid,),
        in_specs=[
            pl.BlockSpec((_EB, 128), lambda i: (i, 0)),
            pl.BlockSpec((_EB, 128), lambda i: (i, 0)),
            full((_NRB, _DH)), full((1, _DH)),
            full((_DH, _DH)), full((1, _DH)),
            full((_DH, _DH)), full((1, _DH)),
            full((_DH, _NTP)), full((1, _NTP)),
            full((125, 1)),
        ],
        out_specs=[
            pl.BlockSpec((_EB, _OUTW), lambda i: (i, 0)),
            pl.BlockSpec((_EB, 8), lambda i: (i, 0)),
        ],
        out_shape=[
            jax.ShapeDtypeStruct((_EPAD, _OUTW), jnp.float32),
            jax.ShapeDtypeStruct((_EPAD, 8), jnp.float32),
        ],
    )(g, pj, m0, b0, m1, b1, m2, b2, m3, b3, wvh)


def kernel(x, edge_index, We_s, We_v, M0, b0, M1, b1, M2, b2, M3, b3,
           Wm_s, Wm_v, Wn_s, Wn_v, Wp_s, Wp_v):
    senders = edge_index[0]
    receivers = edge_index[1]
    npad = _EPAD - _E
    sidx3 = jnp.reshape(
        jnp.concatenate([senders, jnp.zeros((npad,), jnp.int32)]),
        (_NW, _NCHUNK, _CH))
    ridx3 = jnp.reshape(
        jnp.concatenate([receivers, jnp.full((npad,), _N, jnp.int32)]),
        (_NW, _NCHUNK, _CH))
    zeros_acc = jnp.zeros((_NACC, _OUTW), jnp.float32)
    T = We_s.shape[0]
    for t in range(T):
        pos = x[:, :3]
        sc = x[:, 3:]
        wev = We_v[t][0, 0]
        wv = Wm_v[t][:, 0]
        colscale = jnp.ones((_NTP,), jnp.float32)
        colscale = colscale.at[251].set(wv[125])
        colscale = colscale.at[252].set(wev / _SQRT3)
        colscale = colscale.at[253].set(1.0 / _SQRT3)
        colscale = colscale.at[254].set(wev * wv[126])
        colscale = colscale.at[255].set(wv[127])
        m0 = M0[t] / math.sqrt(float(_NRB))
        m1 = M1[t] / math.sqrt(float(_DH))
        m2 = M2[t] / math.sqrt(float(_DH))
        m3 = (M3[t] / math.sqrt(float(_DH))) * colscale[None, :]
        b3s = b3[t] * colscale
        wvh = wv[:125][:, None]

        sc_t = sc @ (We_s[t] / math.sqrt(float(_DSC)))
        table = jnp.concatenate([
            jnp.concatenate([pos, sc_t], axis=1),
            jnp.zeros((_NACC - _N, 128), jnp.float32)], axis=0)

        g, pj = _sc_gather(table, sidx3, ridx3)

        eout, uvout = _edge_stage(g, pj, m0, b0[t][None, :], m1, b1[t][None, :],
                                  m2, b2[t][None, :], m3, b3s[None, :], wvh)

        partial = _sc_scatter(eout, ridx3, zeros_acc)
        seg4 = jnp.zeros((_N, 4), jnp.float32).at[receivers].add(
            uvout[:_E, :4])
        acc = (partial[0] + partial[1])[:_N]
        cnt = jnp.maximum(seg4[:, 3:4], 1.0)
        ms = acc / cnt
        mvc = seg4[:, :3] / cnt
        u_s = (ms @ (Wm_s[t] / math.sqrt(128.0))
               + sc @ (Wn_s[t] / math.sqrt(float(_DSC))))
        u_v = mvc / math.sqrt(128.0) + pos * Wn_v[t][0, 0]
        gate = jax.nn.sigmoid(u_s[:, _DSC:_DSC + 1])
        new_sc = jax.nn.gelu(u_s[:, :_DSC])
        new_v = u_v * gate
        sc2 = new_sc @ (Wp_s[t] / math.sqrt(float(_DSC)))
        v2 = new_v * Wp_v[t][0, 0]
        x = jnp.concatenate([v2, sc2], axis=1)
    return x


# speedup vs baseline: 11.0814x; 1.1701x over previous
import functools
import math

import jax
import jax.numpy as jnp
from jax import lax
from jax.experimental import pallas as pl
from jax.experimental.pallas import tpu as pltpu
from jax.experimental.pallas import tpu_sc as plsc

_N = 10000
_E = 320000
_DSC = 125
_DH = 64
_NRB = 4
_NTP = 258

_NC = 2
_NS = 16
_NW = _NC * _NS
_CH = 128
_NCHUNK = 80
_EPW = _CH * _NCHUNK
_EPAD = _EPW * _NW
_EB = 1024
_OUTW = 128
_NACC = 10112
_RPT = _NACC // _NS
_NQ = 320
_QPT8 = _NQ // 8

_SQRT2 = math.sqrt(2.0)
_SQRT3 = math.sqrt(3.0)

_MESH = plsc.VectorSubcoreMesh(core_axis_name="c", subcore_axis_name="s")


def _gather_body(table_ref, sidx_ref, ridx_ref, g_ref, pj_ref,
                 table_sh, sidx_v, ridx_v, rows_v, sem_g):
    cid = lax.axis_index("c")
    sid = lax.axis_index("s")
    wid = sid * _NC + cid
    pltpu.sync_copy(table_ref.at[pl.ds(sid * _RPT, _RPT)],
                    table_sh.at[pl.ds(sid * _RPT, _RPT)])
    plsc.subcore_barrier()
    base = wid * _EPW
    pltpu.sync_copy(sidx_ref.at[wid], sidx_v)
    pltpu.sync_copy(ridx_ref.at[wid], ridx_v)

    def body(j, carry):
        off = base + j * _CH
        pltpu.async_copy(table_sh.at[sidx_v.at[j]], rows_v, sem_g).wait()
        pltpu.sync_copy(rows_v, g_ref.at[pl.ds(off, _CH)])
        pltpu.async_copy(table_sh.at[ridx_v.at[j]], rows_v, sem_g).wait()
        pltpu.sync_copy(rows_v, pj_ref.at[pl.ds(off, _CH)])
        return carry

    lax.fori_loop(0, _NCHUNK, body, 0)


@functools.partial(
    pl.kernel,
    mesh=_MESH,
    out_type=(
        jax.ShapeDtypeStruct((_EPAD, 128), jnp.float32),
        jax.ShapeDtypeStruct((_EPAD, 128), jnp.float32),
    ),
    scratch_types=[
        pltpu.VMEM_SHARED((_NACC, 128), jnp.float32),
        pltpu.VMEM((_NCHUNK, _CH), jnp.int32),
        pltpu.VMEM((_NCHUNK, _CH), jnp.int32),
        pltpu.VMEM((_CH, 128), jnp.float32),
        pltpu.SemaphoreType.DMA,
    ],
)
def _sc_gather(table, sidx3, ridx3, g_out, pj_out,
               table_sh, sidx_v, ridx_v, rows_v, sem_g):
    _gather_body(table, sidx3, ridx3, g_out, pj_out,
                 table_sh, sidx_v, ridx_v, rows_v, sem_g)


def _scatter_body(eout_ref, uvp_ref, ridx_ref, qidx_ref, zeros_ref,
                  out_ref, out2_ref, idx_v, qidx_v, pay_v, acc_sh, acc2_sh,
                  sem):
    cid = lax.axis_index("c")
    sid = lax.axis_index("s")
    wid = sid * _NC + cid
    pltpu.sync_copy(zeros_ref.at[pl.ds(sid * _RPT, _RPT)],
                    acc_sh.at[pl.ds(sid * _RPT, _RPT)])

    @pl.when(sid < 8)
    def _():
        pltpu.sync_copy(zeros_ref.at[pl.ds(sid * _QPT8, _QPT8)],
                        acc2_sh.at[pl.ds(sid * _QPT8, _QPT8)])

    plsc.subcore_barrier()
    pltpu.sync_copy(ridx_ref.at[wid], idx_v)
    pltpu.sync_copy(qidx_ref.at[wid], qidx_v)
    base = wid * _EPW

    def body(j, carry):
        off = base + j * _CH
        pltpu.sync_copy(eout_ref.at[pl.ds(off, _CH)], pay_v)
        pltpu.sync_copy(pay_v, acc_sh.at[idx_v.at[j]], add=True)
        pltpu.sync_copy(uvp_ref.at[pl.ds(off, _CH)], pay_v)
        pltpu.sync_copy(pay_v, acc2_sh.at[qidx_v.at[j]], add=True)
        return carry

    lax.fori_loop(0, _NCHUNK, body, 0)
    plsc.subcore_barrier()
    pltpu.sync_copy(acc_sh.at[pl.ds(sid * _RPT, _RPT)],
                    out_ref.at[cid, pl.ds(sid * _RPT, _RPT)])

    @pl.when(sid < 8)
    def _():
        pltpu.sync_copy(acc2_sh.at[pl.ds(sid * _QPT8, _QPT8)],
                        out2_ref.at[cid, pl.ds(sid * _QPT8, _QPT8)])


@functools.partial(
    pl.kernel,
    mesh=_MESH,
    out_type=(
        jax.ShapeDtypeStruct((_NC, _NACC, _OUTW), jnp.float32),
        jax.ShapeDtypeStruct((_NC, _NQ, _OUTW), jnp.float32),
    ),
    scratch_types=[
        pltpu.VMEM((_NCHUNK, _CH), jnp.int32),
        pltpu.VMEM((_NCHUNK, _CH), jnp.int32),
        pltpu.VMEM((_CH, _OUTW), jnp.float32),
        pltpu.VMEM_SHARED((_NACC, _OUTW), jnp.float32),
        pltpu.VMEM_SHARED((_NQ, _OUTW), jnp.float32),
        pltpu.SemaphoreType.DMA,
    ],
)
def _sc_scatter(eout, uvp, ridx3, qidx3, zeros, partial, partial2,
                idx_v, qidx_v, pay_v, acc_sh, acc2_sh, sem):
    _scatter_body(eout, uvp, ridx3, qidx3, zeros, partial, partial2,
                  idx_v, qidx_v, pay_v, acc_sh, acc2_sh, sem)


def _edge_body(g_ref, pj_ref, oh_ref, m0_ref, b0_ref, m1_ref, b1_ref,
               m2_ref, b2_ref, m3_ref, b3_ref, wvh_ref, tmat_ref, rmat_ref,
               out_ref, uv_ref):
    g = g_ref[...]
    pos_i = g[:, 0:3]
    m_s = g[:, 3:128]
    pj = pj_ref[:, 0:3]
    r = pos_i - pj
    d2 = jnp.sum(r * r, axis=1, keepdims=True) + 1e-12
    d = jnp.sqrt(d2)
    y1 = (_SQRT3 / d) * r
    pid = jnp.pi * d
    rad = (_SQRT2 / d) * jnp.concatenate(
        [jnp.sin(pid), jnp.sin(2.0 * pid), jnp.sin(3.0 * pid), jnp.sin(4.0 * pid)],
        axis=1)
    h = jax.nn.gelu(jnp.dot(rad, m0_ref[...], preferred_element_type=jnp.float32) + b0_ref[...])
    h = jax.nn.gelu(jnp.dot(h, m1_ref[...], preferred_element_type=jnp.float32) + b1_ref[...])
    h = jax.nn.gelu(jnp.dot(h, m2_ref[...], preferred_element_type=jnp.float32) + b2_ref[...])
    mix = jnp.dot(h, m3_ref[...], preferred_element_type=jnp.float32) + b3_ref[...]
    es_main = m_s * mix[:, 0:125]
    piy = jnp.sum(pos_i * y1, axis=1, keepdims=True)
    y1y1 = jnp.sum(y1 * y1, axis=1, keepdims=True)
    es126 = piy * mix[:, 252:253]
    es127 = y1y1 * mix[:, 253:254]
    alpha = (jnp.dot(m_s * mix[:, 126:251], wvh_ref[...],
                     preferred_element_type=jnp.float32)
             + mix[:, 251:252] + mix[:, 255:256])
    uv = y1 * alpha + pos_i * mix[:, 254:255]
    out_ref[...] = jnp.concatenate(
        [es_main, mix[:, 125:126], es126, es127], axis=1)
    uvcnt = jnp.concatenate([uv, jnp.ones((_EB, 1), jnp.float32)], axis=1)
    uv_ref[...] = (
        jnp.dot(uvcnt, tmat_ref[...], preferred_element_type=jnp.float32)
        * jnp.dot(oh_ref[...], rmat_ref[...],
                  preferred_element_type=jnp.float32))


def _edge_stage(g, pj, oh, m0, b0, m1, b1, m2, b2, m3, b3, wvh, tmat, rmat):
    grid = _EPAD // _EB
    full = lambda shape: pl.BlockSpec(shape, lambda i: (0, 0))
    return pl.pallas_call(
        _edge_body,
        grid=(grid,),
        in_specs=[
            pl.BlockSpec((_EB, 128), lambda i: (i, 0)),
            pl.BlockSpec((_EB, 128), lambda i: (i, 0)),
            pl.BlockSpec((_EB, 32), lambda i: (i, 0)),
            full((_NRB, _DH)), full((1, _DH)),
            full((_DH, _DH)), full((1, _DH)),
            full((_DH, _DH)), full((1, _DH)),
            full((_DH, _NTP)), full((1, _NTP)),
            full((125, 1)), full((4, 128)), full((32, 128)),
        ],
        out_specs=[
            pl.BlockSpec((_EB, _OUTW), lambda i: (i, 0)),
            pl.BlockSpec((_EB, _OUTW), lambda i: (i, 0)),
        ],
        out_shape=[
            jax.ShapeDtypeStruct((_EPAD, _OUTW), jnp.float32),
            jax.ShapeDtypeStruct((_EPAD, _OUTW), jnp.float32),
        ],
    )(g, pj, oh, m0, b0, m1, b1, m2, b2, m3, b3, wvh, tmat, rmat)


def kernel(x, edge_index, We_s, We_v, M0, b0, M1, b1, M2, b2, M3, b3,
           Wm_s, Wm_v, Wn_s, Wn_v, Wp_s, Wp_v):
    senders = edge_index[0]
    receivers = edge_index[1]
    npad = _EPAD - _E
    sidx3 = jnp.reshape(
        jnp.concatenate([senders, jnp.zeros((npad,), jnp.int32)]),
        (_NW, _NCHUNK, _CH))
    rec_pad = jnp.concatenate([receivers, jnp.full((npad,), _N, jnp.int32)])
    ridx3 = jnp.reshape(rec_pad, (_NW, _NCHUNK, _CH))
    qidx3 = jnp.reshape(rec_pad // 32, (_NW, _NCHUNK, _CH))
    slot = rec_pad % 32
    oh = (slot[:, None] == jnp.arange(32)[None, :]).astype(jnp.float32)
    c128 = jnp.arange(128)
    tmat = (c128[None, :] % 4 == jnp.arange(4)[:, None]).astype(jnp.float32)
    rmat = (c128[None, :] // 4 == jnp.arange(32)[:, None]).astype(jnp.float32)
    zeros_acc = jnp.zeros((_NACC, _OUTW), jnp.float32)
    T = We_s.shape[0]
    for t in range(T):
        pos = x[:, :3]
        sc = x[:, 3:]
        wev = We_v[t][0, 0]
        wv = Wm_v[t][:, 0]
        colscale = jnp.ones((_NTP,), jnp.float32)
        colscale = colscale.at[251].set(wv[125])
        colscale = colscale.at[252].set(wev / _SQRT3)
        colscale = colscale.at[253].set(1.0 / _SQRT3)
        colscale = colscale.at[254].set(wev * wv[126])
        colscale = colscale.at[255].set(wv[127])
        m0 = M0[t] / math.sqrt(float(_NRB))
        m1 = M1[t] / math.sqrt(float(_DH))
        m2 = M2[t] / math.sqrt(float(_DH))
        m3 = (M3[t] / math.sqrt(float(_DH))) * colscale[None, :]
        b3s = b3[t] * colscale
        wvh = wv[:125][:, None]

        sc_t = sc @ (We_s[t] / math.sqrt(float(_DSC)))
        table = jnp.concatenate([
            jnp.concatenate([pos, sc_t], axis=1),
            jnp.zeros((_NACC - _N, 128), jnp.float32)], axis=0)

        g, pj = _sc_gather(table, sidx3, ridx3)

        eout, uvp = _edge_stage(g, pj, oh, m0, b0[t][None, :],
                                m1, b1[t][None, :], m2, b2[t][None, :],
                                m3, b3s[None, :], wvh, tmat, rmat)

        partial, partial2 = _sc_scatter(eout, uvp, ridx3, qidx3, zeros_acc)
        seg4 = jnp.reshape(partial2[0] + partial2[1], (_NQ * 32, 4))[:_N]
        acc = (partial[0] + partial[1])[:_N]
        cnt = jnp.maximum(seg4[:, 3:4], 1.0)
        ms = acc / cnt
        mvc = seg4[:, :3] / cnt
        u_s = (ms @ (Wm_s[t] / math.sqrt(128.0))
               + sc @ (Wn_s[t] / math.sqrt(float(_DSC))))
        u_v = mvc / math.sqrt(128.0) + pos * Wn_v[t][0, 0]
        gate = jax.nn.sigmoid(u_s[:, _DSC:_DSC + 1])
        new_sc = jax.nn.gelu(u_s[:, :_DSC])
        new_v = u_v * gate
        sc2 = new_sc @ (Wp_s[t] / math.sqrt(float(_DSC)))
        v2 = new_v * Wp_v[t][0, 0]
        x = jnp.concatenate([v2, sc2], axis=1)
    return x
